# Initial kernel scaffold; baseline (speedup 1.0000x reference)
import jax
import jax.numpy as jnp
from jax import lax
from jax.experimental import pallas as pl
from jax.experimental.pallas import tpu as pltpu
from jax.experimental.pallas import tpu_sc as plsc

NUM_FEATURES = 26
VOCAB = 100000
DIM = 32
BATCH = 16384

_INFO = plsc.get_sparse_core_info()
_NC = _INFO.num_cores
_NS = _INFO.num_subcores
_NW = _NC * _NS
_L = 16

_TOT = BATCH * NUM_FEATURES
_PER_W = _TOT // _NW
_CHUNK = 1664
_N_CHUNKS = _PER_W // _CHUNK
_GROUPS = _CHUNK // _L


def _body(feat_hbm, table_hbm, out_hbm, idx_v, rows_v, sem):
    wid = lax.axis_index("s") * _NC + lax.axis_index("c")
    base = wid * _PER_W
    lanes = lax.iota(jnp.int32, _L)

    def chunk_body(c, _):
        off = base + c * _CHUNK
        pltpu.sync_copy(feat_hbm.at[pl.ds(off, _CHUNK)], idx_v)

        def add_offsets(g, _):
            pos = off + g * _L + lanes
            feat = lax.rem(pos, NUM_FEATURES)
            idx_v[pl.ds(g * _L, _L)] = idx_v[pl.ds(g * _L, _L)] + feat * VOCAB
            return 0

        lax.fori_loop(0, _GROUPS, add_offsets, 0, unroll=4)

        pltpu.async_copy(table_hbm.at[idx_v], rows_v, sem).wait()
        pltpu.sync_copy(rows_v, out_hbm.at[pl.ds(off, _CHUNK)])
        return 0

    lax.fori_loop(0, _N_CHUNKS, chunk_body, 0)


@jax.jit
def _gather(feats_flat, table_flat):
    mesh = plsc.VectorSubcoreMesh(core_axis_name="c", subcore_axis_name="s")
    return pl.kernel(
        _body,
        out_type=jax.ShapeDtypeStruct((_TOT, DIM), jnp.float32),
        mesh=mesh,
        scratch_types=[
            pltpu.VMEM((_CHUNK,), jnp.int32),
            pltpu.VMEM((_CHUNK, DIM), jnp.float32),
            pltpu.SemaphoreType.DMA,
        ],
    )(feats_flat, table_flat)


def kernel(features, tables):
    feats_flat = features.reshape(-1).astype(jnp.int32)
    table_flat = tables.reshape(NUM_FEATURES * VOCAB, DIM)
    out = _gather(feats_flat, table_flat)
    return out.reshape(BATCH, NUM_FEATURES * DIM)


# speedup vs baseline: 1.2089x; 1.2089x over previous
import jax
import jax.numpy as jnp
from jax import lax
from jax.experimental import pallas as pl
from jax.experimental.pallas import tpu as pltpu
from jax.experimental.pallas import tpu_sc as plsc

NUM_FEATURES = 26
VOCAB = 100000
DIM = 32
BATCH = 16384

_INFO = plsc.get_sparse_core_info()
_NC = _INFO.num_cores
_NS = _INFO.num_subcores
_NW = _NC * _NS
_L = 16

_TOT = BATCH * NUM_FEATURES
_PER_W = _TOT // _NW
_CHUNK = 1664
_N_CHUNKS = _PER_W // _CHUNK
_GROUPS = _CHUNK // _L


def _body(feat_hbm, table_hbm, out_hbm, idx_v, rows_v, sem):
    wid = lax.axis_index("s") * _NC + lax.axis_index("c")
    base = wid * _PER_W
    lanes = lax.iota(jnp.int32, _L)

    def chunk_body(c, _):
        off = base + c * _CHUNK
        pltpu.sync_copy(feat_hbm.at[pl.ds(off, _CHUNK)], idx_v)

        def add_offsets(g, _):
            pos = off + g * _L + lanes
            feat = lax.rem(pos, NUM_FEATURES)
            idx_v[pl.ds(g * _L, _L)] = idx_v[pl.ds(g * _L, _L)] + feat * VOCAB
            return 0

        lax.fori_loop(0, _GROUPS, add_offsets, 0, unroll=4)

        pltpu.async_copy(table_hbm.at[idx_v], rows_v, sem).wait()
        pltpu.sync_copy(rows_v, out_hbm.at[pl.ds(off, _CHUNK)])
        return 0

    lax.fori_loop(0, _N_CHUNKS, chunk_body, 0)


@jax.jit
def _gather(feats_flat, table_flat):
    mesh = plsc.VectorSubcoreMesh(core_axis_name="c", subcore_axis_name="s")
    return pl.kernel(
        _body,
        out_type=jax.ShapeDtypeStruct((_TOT, DIM), jnp.float32),
        mesh=mesh,
        scratch_types=[
            pltpu.VMEM((_CHUNK,), jnp.int32),
            pltpu.VMEM((_CHUNK, DIM), jnp.float32),
            pltpu.SemaphoreType.DMA,
        ],
        compiler_params=pltpu.CompilerParams(use_tc_tiling_on_sc=False),
    )(feats_flat, table_flat)


def kernel(features, tables):
    feats_flat = features.reshape(-1).astype(jnp.int32)
    table_flat = tables.reshape(NUM_FEATURES * VOCAB, DIM)
    out = _gather(feats_flat, table_flat)
    return out.reshape(BATCH, NUM_FEATURES * DIM)
